# baseline (speedup 1.0000x reference)
import functools

import jax
import jax.numpy as jnp
from jax import lax
from jax.experimental import pallas as pl
from jax.experimental.pallas import tpu as pltpu
from jax.experimental.pallas import tpu_sc as plsc

N = 10000
E = 320000
IN_DIM = 128
HID = 16
NCLS = 8

NC = 2
NS = 16
NW = NC * NS
CH = 128
K = 80
NB = 4
SLOTS = 4
KE0, KE1 = 48, 112
KD0, KD1 = 64, 96
K_MAX = max(KE0, KE1, KD0, KD1)
E_PAD = NW * K * CH
N_PAD = 10112
ZR = N_PAD // NS
OR_ = N_PAD // NS
BN = 2000
G = N // BN

_mesh = plsc.VectorSubcoreMesh(core_axis_name="c", subcore_axis_name="s")


def _zero_acc(zv, acc, s):
    def _fill(i, carry):
        zv[i] = jnp.zeros((HID,), jnp.float32)
        return carry

    lax.fori_loop(0, ZR, _fill, None)
    pltpu.sync_copy(zv, acc.at[pl.ds(s * ZR, ZR)])


@functools.partial(
    pl.kernel,
    out_type=jax.ShapeDtypeStruct((NC * N_PAD, HID), jnp.float32),
    mesh=_mesh,
    scratch_types=[
        pltpu.VMEM((K_MAX, CH), jnp.int32),
        pltpu.VMEM((CH, HID), jnp.float32),
        pltpu.VMEM((ZR, HID), jnp.float32),
        pltpu.VMEM_SHARED((N_PAD, HID), jnp.float32),
        pltpu.SemaphoreType.DMA,
    ],
    compiler_params=pltpu.CompilerParams(use_tc_tiling_on_sc=False),
)
def _sc_degree(dst_hbm, out_hbm, dstv, ones_v, zv, acc, sem):
    c = lax.axis_index("c")
    s = lax.axis_index("s")

    _zero_acc(zv, acc, s)

    def _fill1(i, carry):
        ones_v[i] = jnp.ones((HID,), jnp.float32)
        return carry

    lax.fori_loop(0, CH, _fill1, None)

    @pl.when(c == 0)
    def _():
        pltpu.sync_copy(dst_hbm.at[pl.ds(s * KD0, KD0)], dstv.at[pl.ds(0, KD0)])

    @pl.when(c == 1)
    def _():
        pltpu.sync_copy(dst_hbm.at[pl.ds(NS * KD0 + s * KD1, KD1)],
                        dstv.at[pl.ds(0, KD1)])

    plsc.subcore_barrier()
    k = jnp.where(c == 0, KD0, KD1)

    def _chunk(j, carry):
        pltpu.async_copy(ones_v, acc.at[dstv.at[j]], sem, add=True)
        return carry

    lax.fori_loop(0, k, _chunk, None)

    def _drain(j, carry):
        pltpu.make_async_copy(out_hbm.at[pl.ds(0, CH)], ones_v, sem).wait()
        return carry

    lax.fori_loop(0, k, _drain, None)

    plsc.subcore_barrier()
    pltpu.sync_copy(acc.at[pl.ds(s * OR_, OR_)],
                    out_hbm.at[pl.ds(c * N_PAD + s * OR_, OR_)])


@functools.partial(
    pl.kernel,
    out_type=jax.ShapeDtypeStruct((NC * N_PAD, HID), jnp.float32),
    mesh=_mesh,
    scratch_types=[
        pltpu.VMEM((K_MAX, CH), jnp.int32),
        pltpu.VMEM((K_MAX, CH), jnp.int32),
        pltpu.VMEM((NB, CH, HID), jnp.float32),
        pltpu.VMEM((NB, CH, HID), jnp.float32),
        pltpu.VMEM((NB, CH, HID), jnp.float32),
        pltpu.VMEM((NB, CH, HID), jnp.float32),
        pltpu.VMEM((ZR, HID), jnp.float32),
        pltpu.VMEM_SHARED((N_PAD, HID), jnp.float32),
        pltpu.SemaphoreType.DMA,
        pltpu.SemaphoreType.DMA,
        pltpu.SemaphoreType.DMA,
        pltpu.SemaphoreType.DMA,
        pltpu.SemaphoreType.DMA,
        pltpu.SemaphoreType.DMA,
        pltpu.SemaphoreType.DMA,
        pltpu.SemaphoreType.DMA,
    ],
    compiler_params=pltpu.CompilerParams(use_tc_tiling_on_sc=False),
)
def _sc_edge_agg(g_hbm, src_hbm, dst_hbm, out_hbm, srcv, dstv,
                 r0, r1, r2, r3, zv, acc,
                 sg0, sg1, sg2, sg3, ss0, ss1, ss2, ss3):
    c = lax.axis_index("c")
    s = lax.axis_index("s")
    rows = (r0, r1, r2, r3)
    semg = (sg0, sg1, sg2, sg3)
    sems = (ss0, ss1, ss2, ss3)

    with jax.named_scope("agg_zero"):
        _zero_acc(zv, acc, s)

    with jax.named_scope("agg_loadidx"):
        @pl.when(c == 0)
        def _():
            pltpu.sync_copy(src_hbm.at[pl.ds(s * KE0, KE0)], srcv.at[pl.ds(0, KE0)])
            pltpu.sync_copy(dst_hbm.at[pl.ds(s * KE0, KE0)], dstv.at[pl.ds(0, KE0)])

        @pl.when(c == 1)
        def _():
            base = NS * KE0 + s * KE1
            pltpu.sync_copy(src_hbm.at[pl.ds(base, KE1)], srcv.at[pl.ds(0, KE1)])
            pltpu.sync_copy(dst_hbm.at[pl.ds(base, KE1)], dstv.at[pl.ds(0, KE1)])

    with jax.named_scope("agg_barrier1"):
        plsc.subcore_barrier()

    ng4 = jnp.where(c == 0, KE0 // (NB * 4), KE1 // (NB * 4))

    def _fire_g(g, p):
        for b in range(NB):
            pltpu.async_copy(g_hbm.at[srcv.at[g * NB + b]], rows[p].at[b], semg[p])

    def _wait_g(p):
        for b in range(NB):
            pltpu.make_async_copy(g_hbm.at[pl.ds(0, CH)], rows[p].at[b],
                                  semg[p]).wait()

    def _fire_s(g, p):
        for b in range(NB):
            pltpu.async_copy(rows[p].at[b], acc.at[dstv.at[g * NB + b]],
                             sems[p], add=True)

    def _wait_s(p):
        for b in range(NB):
            pltpu.make_async_copy(g_hbm.at[pl.ds(0, CH)], rows[p].at[b],
                                  sems[p]).wait()

    with jax.named_scope("agg_prologue"):
        _fire_g(0, 0)
        _fire_g(1, 1)

    def _body(t, carry):
        for p in range(4):
            g = 4 * t + p
            q = (p + 2) % 4
            _wait_g(p)
            _fire_s(g, p)
            if p >= 2:
                _wait_s(q)

                @pl.when(t < ng4 - 1)
                def _():
                    _fire_g(g + 2, q)
            else:
                @pl.when(t > 0)
                def _():
                    _wait_s(q)

                _fire_g(g + 2, q)
        return carry

    with jax.named_scope("agg_mainloop"):
        lax.fori_loop(0, ng4, _body, None)
        _wait_s(2)
        _wait_s(3)

    with jax.named_scope("agg_barrier2"):
        plsc.subcore_barrier()
    with jax.named_scope("agg_copyout"):
        pltpu.sync_copy(acc.at[pl.ds(s * OR_, OR_)],
                        out_hbm.at[pl.ds(c * N_PAD + s * OR_, OR_)])


def _tc_mm0_body(x_ref, w_ref, h_ref):
    h_ref[...] = jnp.dot(x_ref[...], w_ref[...], preferred_element_type=jnp.float32)


def _tc_mm0(x, W0):
    return pl.pallas_call(
        _tc_mm0_body,
        grid=(G,),
        in_specs=[
            pl.BlockSpec((BN, IN_DIM), lambda i: (i, 0)),
            pl.BlockSpec((IN_DIM, HID), lambda i: (0, 0)),
        ],
        out_specs=pl.BlockSpec((BN, HID), lambda i: (i, 0)),
        out_shape=jax.ShapeDtypeStruct((N, HID), jnp.float32),
    )(x, W0)


def _tc_scale_body(h_ref, degp_ref, g_ref, dinv_ref):
    deg = degp_ref[0] + degp_ref[1] + 1.0
    dinv = lax.rsqrt(jnp.maximum(deg, 1.0))
    g_ref[...] = h_ref[...] * dinv
    dinv_ref[...] = dinv


def _tc_scale(h, degp):
    return pl.pallas_call(
        _tc_scale_body,
        grid=(G,),
        in_specs=[
            pl.BlockSpec((BN, HID), lambda i: (i, 0)),
            pl.BlockSpec((2, BN, HID), lambda i: (0, i, 0)),
        ],
        out_specs=[
            pl.BlockSpec((BN, HID), lambda i: (i, 0)),
            pl.BlockSpec((BN, HID), lambda i: (i, 0)),
        ],
        out_shape=[
            jax.ShapeDtypeStruct((N, HID), jnp.float32),
            jax.ShapeDtypeStruct((N, HID), jnp.float32),
        ],
    )(h, degp)


def _tc_mid_body(p_ref, g_ref, dinv_ref, w_ref, b_ref, gn_ref):
    dinv = dinv_ref[...]
    z = dinv * (p_ref[0] + p_ref[1] + g_ref[...]) + b_ref[...]
    t = jnp.tanh(z)
    gn_ref[...] = jnp.dot(t, w_ref[...], preferred_element_type=jnp.float32) * dinv


def _tc_mid(p, g, dinv, W, b):
    return pl.pallas_call(
        _tc_mid_body,
        grid=(G,),
        in_specs=[
            pl.BlockSpec((2, BN, HID), lambda i: (0, i, 0)),
            pl.BlockSpec((BN, HID), lambda i: (i, 0)),
            pl.BlockSpec((BN, HID), lambda i: (i, 0)),
            pl.BlockSpec((HID, HID), lambda i: (0, 0)),
            pl.BlockSpec((1, HID), lambda i: (0, 0)),
        ],
        out_specs=pl.BlockSpec((BN, HID), lambda i: (i, 0)),
        out_shape=jax.ShapeDtypeStruct((N, HID), jnp.float32),
    )(p, g, dinv, W, b)


def _tc_fin_body(p_ref, g_ref, dinv_ref, b_ref, wc_ref, bc_ref, out_ref, emb_ref):
    dinv = dinv_ref[...]
    z = dinv * (p_ref[0] + p_ref[1] + g_ref[...]) + b_ref[...]
    emb = jnp.tanh(z)
    emb_ref[...] = emb
    out_ref[...] = jnp.dot(emb, wc_ref[...], preferred_element_type=jnp.float32) + bc_ref[...]


def _tc_fin(p, g, dinv, b2, Wc, bc):
    return pl.pallas_call(
        _tc_fin_body,
        grid=(G,),
        in_specs=[
            pl.BlockSpec((2, BN, HID), lambda i: (0, i, 0)),
            pl.BlockSpec((BN, HID), lambda i: (i, 0)),
            pl.BlockSpec((BN, HID), lambda i: (i, 0)),
            pl.BlockSpec((1, HID), lambda i: (0, 0)),
            pl.BlockSpec((HID, NCLS), lambda i: (0, 0)),
            pl.BlockSpec((1, NCLS), lambda i: (0, 0)),
        ],
        out_specs=[
            pl.BlockSpec((BN, NCLS), lambda i: (i, 0)),
            pl.BlockSpec((BN, HID), lambda i: (i, 0)),
        ],
        out_shape=[
            jax.ShapeDtypeStruct((N, NCLS), jnp.float32),
            jax.ShapeDtypeStruct((N, HID), jnp.float32),
        ],
    )(p, g, dinv, b2, Wc, bc)


def kernel(x, edge_index, W0, b0, W1, b1, W2, b2, Wc, bc):
    src = edge_index[0]
    dst = edge_index[1]
    pad = E_PAD - E
    srcp = jnp.concatenate([src, jnp.zeros((pad,), jnp.int32)]).reshape(NW * K, CH)
    dstp = jnp.concatenate([dst, jnp.full((pad,), N, jnp.int32)]).reshape(NW * K, CH)

    h0 = _tc_mm0(x, W0)
    degp = _sc_degree(dstp).reshape(2, N_PAD, HID)
    g0, dinv = _tc_scale(h0, degp)

    p0 = _sc_edge_agg(g0, srcp, dstp).reshape(2, N_PAD, HID)
    g1 = _tc_mid(p0, g0, dinv, W1, b0.reshape(1, HID))

    p1 = _sc_edge_agg(g1, srcp, dstp).reshape(2, N_PAD, HID)
    g2 = _tc_mid(p1, g1, dinv, W2, b1.reshape(1, HID))

    p2 = _sc_edge_agg(g2, srcp, dstp).reshape(2, N_PAD, HID)
    out, emb = _tc_fin(p2, g2, dinv, b2.reshape(1, HID), Wc, bc.reshape(1, NCLS))
    return (out, emb)


# speedup vs baseline: 1.6774x; 1.6774x over previous
import functools

import jax
import jax.numpy as jnp
from jax import lax
from jax.experimental import pallas as pl
from jax.experimental.pallas import tpu as pltpu
from jax.experimental.pallas import tpu_sc as plsc

N = 10000
E = 320000
IN_DIM = 128
HID = 16
NCLS = 8

NC = 2
NS = 16
NW = NC * NS
CH = 128
K = 80
NB = 4
SLOTS = 4
KE0, KE1 = 80, 80
KD0, KD1 = 80, 80
K_MAX = max(KE0, KE1, KD0, KD1)
E_PAD = NW * K * CH
N_PAD = 10112
ZR = N_PAD // NS
OR_ = N_PAD // NS
BN = 2000
G = N // BN

_mesh = plsc.VectorSubcoreMesh(core_axis_name="c", subcore_axis_name="s")


def _zero_acc(zv, acc, s):
    def _fill(i, carry):
        zv[i] = jnp.zeros((HID,), jnp.float32)
        return carry

    lax.fori_loop(0, ZR, _fill, None)
    pltpu.sync_copy(zv, acc.at[pl.ds(s * ZR, ZR)])


@functools.partial(
    pl.kernel,
    out_type=jax.ShapeDtypeStruct((NC * N_PAD, HID), jnp.float32),
    mesh=_mesh,
    scratch_types=[
        pltpu.VMEM((K_MAX, CH), jnp.int32),
        pltpu.VMEM((CH, HID), jnp.float32),
        pltpu.VMEM((ZR, HID), jnp.float32),
        pltpu.VMEM_SHARED((N_PAD, HID), jnp.float32),
        pltpu.SemaphoreType.DMA,
    ],
    compiler_params=pltpu.CompilerParams(use_tc_tiling_on_sc=False),
)
def _sc_degree(dst_hbm, out_hbm, dstv, ones_v, zv, acc, sem):
    c = lax.axis_index("c")
    s = lax.axis_index("s")

    _zero_acc(zv, acc, s)

    def _fill1(i, carry):
        ones_v[i] = jnp.ones((HID,), jnp.float32)
        return carry

    lax.fori_loop(0, CH, _fill1, None)

    @pl.when(c == 0)
    def _():
        pltpu.sync_copy(dst_hbm.at[pl.ds(s * KD0, KD0)], dstv.at[pl.ds(0, KD0)])

    @pl.when(c == 1)
    def _():
        pltpu.sync_copy(dst_hbm.at[pl.ds(NS * KD0 + s * KD1, KD1)],
                        dstv.at[pl.ds(0, KD1)])

    plsc.subcore_barrier()
    k = jnp.where(c == 0, KD0, KD1)

    def _chunk(j, carry):
        pltpu.async_copy(ones_v, acc.at[dstv.at[j]], sem, add=True)
        return carry

    lax.fori_loop(0, k, _chunk, None)

    def _drain(j, carry):
        pltpu.make_async_copy(out_hbm.at[pl.ds(0, CH)], ones_v, sem).wait()
        return carry

    lax.fori_loop(0, k, _drain, None)

    plsc.subcore_barrier()
    pltpu.sync_copy(acc.at[pl.ds(s * OR_, OR_)],
                    out_hbm.at[pl.ds(c * N_PAD + s * OR_, OR_)])


@functools.partial(
    pl.kernel,
    out_type=jax.ShapeDtypeStruct((NC * N_PAD, HID), jnp.float32),
    mesh=_mesh,
    scratch_types=[
        pltpu.VMEM((K_MAX, CH), jnp.int32),
        pltpu.VMEM((K_MAX, CH), jnp.int32),
        pltpu.VMEM((NB, CH, HID), jnp.float32),
        pltpu.VMEM((NB, CH, HID), jnp.float32),
        pltpu.VMEM((NB, CH, HID), jnp.float32),
        pltpu.VMEM((NB, CH, HID), jnp.float32),
        pltpu.VMEM((ZR, HID), jnp.float32),
        pltpu.VMEM_SHARED((N_PAD, HID), jnp.float32),
        pltpu.SemaphoreType.DMA,
        pltpu.SemaphoreType.DMA,
        pltpu.SemaphoreType.DMA,
        pltpu.SemaphoreType.DMA,
        pltpu.SemaphoreType.DMA,
        pltpu.SemaphoreType.DMA,
        pltpu.SemaphoreType.DMA,
        pltpu.SemaphoreType.DMA,
    ],
    compiler_params=pltpu.CompilerParams(use_tc_tiling_on_sc=False),
)
def _sc_edge_agg(g_hbm, src_hbm, dst_hbm, out_hbm, srcv, dstv,
                 r0, r1, r2, r3, zv, acc,
                 sg0, sg1, sg2, sg3, ss0, ss1, ss2, ss3):
    c = lax.axis_index("c")
    s = lax.axis_index("s")
    rows = (r0, r1, r2, r3)
    semg = (sg0, sg1, sg2, sg3)
    sems = (ss0, ss1, ss2, ss3)

    with jax.named_scope("agg_zero"):
        _zero_acc(zv, acc, s)

    with jax.named_scope("agg_loadidx"):
        @pl.when(c == 0)
        def _():
            pltpu.sync_copy(src_hbm.at[pl.ds(s * KE0, KE0)], srcv.at[pl.ds(0, KE0)])
            pltpu.sync_copy(dst_hbm.at[pl.ds(s * KE0, KE0)], dstv.at[pl.ds(0, KE0)])

        @pl.when(c == 1)
        def _():
            base = NS * KE0 + s * KE1
            pltpu.sync_copy(src_hbm.at[pl.ds(base, KE1)], srcv.at[pl.ds(0, KE1)])
            pltpu.sync_copy(dst_hbm.at[pl.ds(base, KE1)], dstv.at[pl.ds(0, KE1)])

    with jax.named_scope("agg_barrier1"):
        plsc.subcore_barrier()

    ng4 = jnp.where(c == 0, KE0 // (NB * 4), KE1 // (NB * 4))

    def _fire_g(g, p):
        for b in range(NB):
            pltpu.async_copy(g_hbm.at[srcv.at[g * NB + b]], rows[p].at[b], semg[p])

    def _wait_g(p):
        for b in range(NB):
            pltpu.make_async_copy(g_hbm.at[pl.ds(0, CH)], rows[p].at[b],
                                  semg[p]).wait()

    def _fire_s(g, p):
        for b in range(NB):
            pltpu.async_copy(rows[p].at[b], acc.at[dstv.at[g * NB + b]],
                             sems[p], add=True)

    def _wait_s(p):
        for b in range(NB):
            pltpu.make_async_copy(g_hbm.at[pl.ds(0, CH)], rows[p].at[b],
                                  sems[p]).wait()

    with jax.named_scope("agg_prologue"):
        _fire_g(0, 0)
        _fire_g(1, 1)

    def _body(t, carry):
        for p in range(4):
            g = 4 * t + p
            q = (p + 2) % 4
            _wait_g(p)
            _fire_s(g, p)
            if p >= 2:
                _wait_s(q)

                @pl.when(t < ng4 - 1)
                def _():
                    _fire_g(g + 2, q)
            else:
                @pl.when(t > 0)
                def _():
                    _wait_s(q)

                _fire_g(g + 2, q)
        return carry

    with jax.named_scope("agg_mainloop"):
        lax.fori_loop(0, ng4, _body, None)
        _wait_s(2)
        _wait_s(3)

    with jax.named_scope("agg_barrier2"):
        plsc.subcore_barrier()
    with jax.named_scope("agg_copyout"):
        pltpu.sync_copy(acc.at[pl.ds(s * OR_, OR_)],
                        out_hbm.at[pl.ds(c * N_PAD + s * OR_, OR_)])


def _tc_mm0_body(x_ref, w_ref, h_ref):
    h_ref[...] = jnp.dot(x_ref[...], w_ref[...], preferred_element_type=jnp.float32)


def _tc_mm0(x, W0):
    return pl.pallas_call(
        _tc_mm0_body,
        grid=(G,),
        in_specs=[
            pl.BlockSpec((BN, IN_DIM), lambda i: (i, 0)),
            pl.BlockSpec((IN_DIM, HID), lambda i: (0, 0)),
        ],
        out_specs=pl.BlockSpec((BN, HID), lambda i: (i, 0)),
        out_shape=jax.ShapeDtypeStruct((N, HID), jnp.float32),
    )(x, W0)


def _tc_scale_body(h_ref, degp_ref, g_ref, dinv_ref):
    deg = degp_ref[0] + degp_ref[1] + 1.0
    dinv = lax.rsqrt(jnp.maximum(deg, 1.0))
    g_ref[...] = h_ref[...] * dinv
    dinv_ref[...] = dinv


def _tc_scale(h, degp):
    return pl.pallas_call(
        _tc_scale_body,
        grid=(G,),
        in_specs=[
            pl.BlockSpec((BN, HID), lambda i: (i, 0)),
            pl.BlockSpec((2, BN, HID), lambda i: (0, i, 0)),
        ],
        out_specs=[
            pl.BlockSpec((BN, HID), lambda i: (i, 0)),
            pl.BlockSpec((BN, HID), lambda i: (i, 0)),
        ],
        out_shape=[
            jax.ShapeDtypeStruct((N, HID), jnp.float32),
            jax.ShapeDtypeStruct((N, HID), jnp.float32),
        ],
    )(h, degp)


def _tc_mid_body(p_ref, g_ref, dinv_ref, w_ref, b_ref, gn_ref):
    dinv = dinv_ref[...]
    z = dinv * (p_ref[0] + p_ref[1] + g_ref[...]) + b_ref[...]
    t = jnp.tanh(z)
    gn_ref[...] = jnp.dot(t, w_ref[...], preferred_element_type=jnp.float32) * dinv


def _tc_mid(p, g, dinv, W, b):
    return pl.pallas_call(
        _tc_mid_body,
        grid=(G,),
        in_specs=[
            pl.BlockSpec((2, BN, HID), lambda i: (0, i, 0)),
            pl.BlockSpec((BN, HID), lambda i: (i, 0)),
            pl.BlockSpec((BN, HID), lambda i: (i, 0)),
            pl.BlockSpec((HID, HID), lambda i: (0, 0)),
            pl.BlockSpec((1, HID), lambda i: (0, 0)),
        ],
        out_specs=pl.BlockSpec((BN, HID), lambda i: (i, 0)),
        out_shape=jax.ShapeDtypeStruct((N, HID), jnp.float32),
    )(p, g, dinv, W, b)


def _tc_fin_body(p_ref, g_ref, dinv_ref, b_ref, wc_ref, bc_ref, out_ref, emb_ref):
    dinv = dinv_ref[...]
    z = dinv * (p_ref[0] + p_ref[1] + g_ref[...]) + b_ref[...]
    emb = jnp.tanh(z)
    emb_ref[...] = emb
    out_ref[...] = jnp.dot(emb, wc_ref[...], preferred_element_type=jnp.float32) + bc_ref[...]


def _tc_fin(p, g, dinv, b2, Wc, bc):
    return pl.pallas_call(
        _tc_fin_body,
        grid=(G,),
        in_specs=[
            pl.BlockSpec((2, BN, HID), lambda i: (0, i, 0)),
            pl.BlockSpec((BN, HID), lambda i: (i, 0)),
            pl.BlockSpec((BN, HID), lambda i: (i, 0)),
            pl.BlockSpec((1, HID), lambda i: (0, 0)),
            pl.BlockSpec((HID, NCLS), lambda i: (0, 0)),
            pl.BlockSpec((1, NCLS), lambda i: (0, 0)),
        ],
        out_specs=[
            pl.BlockSpec((BN, NCLS), lambda i: (i, 0)),
            pl.BlockSpec((BN, HID), lambda i: (i, 0)),
        ],
        out_shape=[
            jax.ShapeDtypeStruct((N, NCLS), jnp.float32),
            jax.ShapeDtypeStruct((N, HID), jnp.float32),
        ],
    )(p, g, dinv, b2, Wc, bc)


def kernel(x, edge_index, W0, b0, W1, b1, W2, b2, Wc, bc):
    src = edge_index[0]
    dst = edge_index[1]
    pad = E_PAD - E
    pad_i = jnp.arange(pad, dtype=jnp.int32)
    srcp = jnp.concatenate([src, pad_i % N]).reshape(NW * K, CH)
    dstp = jnp.concatenate([dst, N + pad_i % (N_PAD - N)]).reshape(NW * K, CH)

    h0 = _tc_mm0(x, W0)
    degp = _sc_degree(dstp).reshape(2, N_PAD, HID)
    g0, dinv = _tc_scale(h0, degp)

    p0 = _sc_edge_agg(g0, srcp, dstp).reshape(2, N_PAD, HID)
    g1 = _tc_mid(p0, g0, dinv, W1, b0.reshape(1, HID))

    p1 = _sc_edge_agg(g1, srcp, dstp).reshape(2, N_PAD, HID)
    g2 = _tc_mid(p1, g1, dinv, W2, b1.reshape(1, HID))

    p2 = _sc_edge_agg(g2, srcp, dstp).reshape(2, N_PAD, HID)
    out, emb = _tc_fin(p2, g2, dinv, b2.reshape(1, HID), Wc, bc.reshape(1, NCLS))
    return (out, emb)


# speedup vs baseline: 2.3273x; 1.3875x over previous
import functools

import jax
import jax.numpy as jnp
from jax import lax
from jax.experimental import pallas as pl
from jax.experimental.pallas import tpu as pltpu
from jax.experimental.pallas import tpu_sc as plsc

N = 10000
E = 320000
IN_DIM = 128
HID = 16
NCLS = 8

NC = 2
NS = 16
NW = NC * NS
CH = 128
KM = 72
KT = 8
K = KM + KT
NB = 4
NG4 = K // (NB * 4)
EM = NW * KM * CH
TT = NW * KT
TREAL = (E - EM) // CH
N_PAD = 10112
ZR = N_PAD // NS
OW = N // NS

NP = N // 8

_mesh = plsc.VectorSubcoreMesh(core_axis_name="c", subcore_axis_name="s")


def _zero_acc(zv, acc, s):
    def _fill(i, carry):
        for u in range(8):
            zv[i * 8 + u] = jnp.zeros((HID,), jnp.float32)
        return carry

    lax.fori_loop(0, ZR // 8, _fill, None)
    pltpu.sync_copy(zv, acc.at[pl.ds(s * ZR, ZR)])


@functools.partial(
    pl.kernel,
    out_type=jax.ShapeDtypeStruct((NC * N, HID), jnp.float32),
    mesh=_mesh,
    scratch_types=[
        pltpu.VMEM((K, CH), jnp.int32),
        pltpu.VMEM((CH, HID), jnp.float32),
        pltpu.VMEM((ZR, HID), jnp.float32),
        pltpu.VMEM_SHARED((N_PAD, HID), jnp.float32),
        pltpu.SemaphoreType.DMA,
    ],
    compiler_params=pltpu.CompilerParams(use_tc_tiling_on_sc=False),
)
def _sc_degree(dstm_hbm, dstt_hbm, out_hbm, dstv, ones_v, zv, acc, sem):
    c = lax.axis_index("c")
    s = lax.axis_index("s")
    wid = c * NS + s

    _zero_acc(zv, acc, s)

    def _fill1(i, carry):
        for u in range(8):
            ones_v[i * 8 + u] = jnp.ones((HID,), jnp.float32)
        return carry

    lax.fori_loop(0, CH // 8, _fill1, None)

    pltpu.sync_copy(dstm_hbm.at[pl.ds(wid * KM, KM)], dstv.at[pl.ds(0, KM)])
    pltpu.sync_copy(dstt_hbm.at[pl.ds(wid * KT, KT)], dstv.at[pl.ds(KM, KT)])

    plsc.subcore_barrier()

    def _chunk(j, carry):
        pltpu.async_copy(ones_v, acc.at[dstv.at[j]], sem, add=True)
        return carry

    lax.fori_loop(0, K, _chunk, None)

    def _drain(j, carry):
        pltpu.make_async_copy(out_hbm.at[pl.ds(0, CH)], ones_v, sem).wait()
        return carry

    lax.fori_loop(0, K, _drain, None)

    plsc.subcore_barrier()
    pltpu.sync_copy(acc.at[pl.ds(s * OW, OW)],
                    out_hbm.at[pl.ds(c * N + s * OW, OW)])


@functools.partial(
    pl.kernel,
    out_type=jax.ShapeDtypeStruct((NC * N, HID), jnp.float32),
    mesh=_mesh,
    scratch_types=[
        pltpu.VMEM((K, CH), jnp.int32),
        pltpu.VMEM((K, CH), jnp.int32),
        pltpu.VMEM((NB, CH, HID), jnp.float32),
        pltpu.VMEM((NB, CH, HID), jnp.float32),
        pltpu.VMEM((NB, CH, HID), jnp.float32),
        pltpu.VMEM((NB, CH, HID), jnp.float32),
        pltpu.VMEM((ZR, HID), jnp.float32),
        pltpu.VMEM_SHARED((N_PAD, HID), jnp.float32),
        pltpu.SemaphoreType.DMA,
        pltpu.SemaphoreType.DMA,
        pltpu.SemaphoreType.DMA,
        pltpu.SemaphoreType.DMA,
        pltpu.SemaphoreType.DMA,
        pltpu.SemaphoreType.DMA,
        pltpu.SemaphoreType.DMA,
        pltpu.SemaphoreType.DMA,
    ],
    compiler_params=pltpu.CompilerParams(use_tc_tiling_on_sc=False),
)
def _sc_edge_agg(g_hbm, srcm_hbm, srct_hbm, dstm_hbm, dstt_hbm, out_hbm,
                 srcv, dstv, r0, r1, r2, r3, zv, acc,
                 sg0, sg1, sg2, sg3, ss0, ss1, ss2, ss3):
    c = lax.axis_index("c")
    s = lax.axis_index("s")
    wid = c * NS + s
    rows = (r0, r1, r2, r3)
    semg = (sg0, sg1, sg2, sg3)
    sems = (ss0, ss1, ss2, ss3)

    with jax.named_scope("agg_zero"):
        _zero_acc(zv, acc, s)

    with jax.named_scope("agg_loadidx"):
        pltpu.sync_copy(srcm_hbm.at[pl.ds(wid * KM, KM)], srcv.at[pl.ds(0, KM)])
        pltpu.sync_copy(srct_hbm.at[pl.ds(wid * KT, KT)], srcv.at[pl.ds(KM, KT)])
        pltpu.sync_copy(dstm_hbm.at[pl.ds(wid * KM, KM)], dstv.at[pl.ds(0, KM)])
        pltpu.sync_copy(dstt_hbm.at[pl.ds(wid * KT, KT)], dstv.at[pl.ds(KM, KT)])

    with jax.named_scope("agg_barrier1"):
        plsc.subcore_barrier()

    def _fire_g(g, p):
        for b in range(NB):
            pltpu.async_copy(g_hbm.at[srcv.at[g * NB + b]], rows[p].at[b], semg[p])

    def _wait_g(p):
        for b in range(NB):
            pltpu.make_async_copy(g_hbm.at[pl.ds(0, CH)], rows[p].at[b],
                                  semg[p]).wait()

    def _fire_s(g, p):
        for b in range(NB):
            pltpu.async_copy(rows[p].at[b], acc.at[dstv.at[g * NB + b]],
                             sems[p], add=True)

    def _wait_s(p):
        for b in range(NB):
            pltpu.make_async_copy(g_hbm.at[pl.ds(0, CH)], rows[p].at[b],
                                  sems[p]).wait()

    with jax.named_scope("agg_prologue"):
        _fire_g(0, 0)
        _fire_g(1, 1)

    def _body(t, carry):
        for p in range(4):
            g = 4 * t + p
            q = (p + 2) % 4
            _wait_g(p)
            _fire_s(g, p)
            if p >= 2:
                _wait_s(q)

                @pl.when(t < NG4 - 1)
                def _():
                    _fire_g(g + 2, q)
            else:
                @pl.when(t > 0)
                def _():
                    _wait_s(q)

                _fire_g(g + 2, q)
        return carry

    with jax.named_scope("agg_mainloop"):
        lax.fori_loop(0, NG4, _body, None)
        _wait_s(2)
        _wait_s(3)

    with jax.named_scope("agg_barrier2"):
        plsc.subcore_barrier()
    with jax.named_scope("agg_copyout"):
        pltpu.sync_copy(acc.at[pl.ds(s * OW, OW)],
                        out_hbm.at[pl.ds(c * N + s * OW, OW)])


def _tc_mm0_body(x_ref, w_ref, h_ref):
    h_ref[...] = jnp.dot(x_ref[...], w_ref[...], preferred_element_type=jnp.float32)


def _tc_mm0(xr, W0bd):
    return pl.pallas_call(
        _tc_mm0_body,
        out_shape=jax.ShapeDtypeStruct((NP, 8 * HID), jnp.float32),
    )(xr, W0bd)


def _tc_scale_body(h_ref, degp_ref, g_ref, dinv_ref):
    deg = degp_ref[0] + degp_ref[1] + 1.0
    dinv = lax.rsqrt(jnp.maximum(deg, 1.0))
    g_ref[...] = h_ref[...] * dinv
    dinv_ref[...] = dinv


def _tc_scale(h, degp):
    return pl.pallas_call(
        _tc_scale_body,
        out_shape=[
            jax.ShapeDtypeStruct((NP, 8 * HID), jnp.float32),
            jax.ShapeDtypeStruct((NP, 8 * HID), jnp.float32),
        ],
    )(h, degp)


def _tc_mid_body(p_ref, g_ref, dinv_ref, w_ref, b_ref, gn_ref):
    dinv = dinv_ref[...]
    z = dinv * (p_ref[0] + p_ref[1] + g_ref[...]) + b_ref[...]
    t = jnp.tanh(z)
    gn_ref[...] = jnp.dot(t, w_ref[...], preferred_element_type=jnp.float32) * dinv


def _tc_mid(p, g, dinv, Wbd, bt):
    return pl.pallas_call(
        _tc_mid_body,
        out_shape=jax.ShapeDtypeStruct((NP, 8 * HID), jnp.float32),
    )(p, g, dinv, Wbd, bt)


def _tc_fin_body(p_ref, g_ref, dinv_ref, b_ref, wc_ref, bc_ref, out_ref, emb_ref):
    dinv = dinv_ref[...]
    z = dinv * (p_ref[0] + p_ref[1] + g_ref[...]) + b_ref[...]
    emb = jnp.tanh(z)
    emb_ref[...] = emb
    out_ref[...] = jnp.dot(emb, wc_ref[...], preferred_element_type=jnp.float32) + bc_ref[...]


def _tc_fin(p, g, dinv, b2t, Wcbd, bct):
    return pl.pallas_call(
        _tc_fin_body,
        out_shape=[
            jax.ShapeDtypeStruct((NP, 8 * NCLS), jnp.float32),
            jax.ShapeDtypeStruct((NP, 8 * HID), jnp.float32),
        ],
    )(p, g, dinv, b2t, Wcbd, bct)


def kernel(x, edge_index, W0, b0, W1, b1, W2, b2, Wc, bc):
    src = edge_index[0]
    dst = edge_index[1]
    eye8 = jnp.eye(8, dtype=jnp.float32)

    srcm = src[:EM].reshape(NW * KM, CH)
    dstm = dst[:EM].reshape(NW * KM, CH)
    pad_i = jnp.arange((TT - TREAL) * CH, dtype=jnp.int32)
    srct = jnp.concatenate([src[EM:], pad_i % N]).reshape(TT, CH)
    dstt = jnp.concatenate([dst[EM:], N + pad_i % (N_PAD - N)]).reshape(TT, CH)

    xr = x.reshape(NP, 8 * IN_DIM)
    W0bd = jnp.kron(eye8, W0)
    W1bd = jnp.kron(eye8, W1)
    W2bd = jnp.kron(eye8, W2)
    Wcbd = jnp.kron(eye8, Wc)
    b0t = jnp.tile(b0, 8).reshape(1, 8 * HID)
    b1t = jnp.tile(b1, 8).reshape(1, 8 * HID)
    b2t = jnp.tile(b2, 8).reshape(1, 8 * HID)
    bct = jnp.tile(bc, 8).reshape(1, 8 * NCLS)

    h0 = _tc_mm0(xr, W0bd)
    degp = _sc_degree(dstm, dstt).reshape(2, NP, 8 * HID)
    g0, dinv = _tc_scale(h0, degp)

    p0 = _sc_edge_agg(g0.reshape(N, HID), srcm, srct, dstm, dstt)
    g1 = _tc_mid(p0.reshape(2, NP, 8 * HID), g0, dinv, W1bd, b0t)

    p1 = _sc_edge_agg(g1.reshape(N, HID), srcm, srct, dstm, dstt)
    g2 = _tc_mid(p1.reshape(2, NP, 8 * HID), g1, dinv, W2bd, b1t)

    p2 = _sc_edge_agg(g2.reshape(N, HID), srcm, srct, dstm, dstt)
    out_p, emb_p = _tc_fin(p2.reshape(2, NP, 8 * HID), g2, dinv, b2t, Wcbd, bct)
    return (out_p.reshape(N, NCLS), emb_p.reshape(N, HID))


# speedup vs baseline: 2.5336x; 1.0886x over previous
import functools

import jax
import jax.numpy as jnp
from jax import lax
from jax.experimental import pallas as pl
from jax.experimental.pallas import tpu as pltpu
from jax.experimental.pallas import tpu_sc as plsc

N = 10000
E = 320000
IN_DIM = 128
HID = 16
NCLS = 8

NC = 2
NS = 16
NW = NC * NS
CH = 128
KM = 72
KT = 8
K = KM + KT
NB = 4
NG4 = K // (NB * 4)
EM = NW * KM * CH
TT = NW * KT
TREAL = (E - EM) // CH
N_PAD = 10112
ZR = N_PAD // NS
OW = N_PAD // NS

NP = N_PAD // 8
PADR = NP - N // 8

_mesh = plsc.VectorSubcoreMesh(core_axis_name="c", subcore_axis_name="s")


def _zero_acc(zv, acc, s):
    def _fill(i, carry):
        for u in range(8):
            zv[i * 8 + u] = jnp.zeros((HID,), jnp.float32)
        return carry

    lax.fori_loop(0, ZR // 8, _fill, None)
    pltpu.sync_copy(zv, acc.at[pl.ds(s * ZR, ZR)])


@functools.partial(
    pl.kernel,
    out_type=jax.ShapeDtypeStruct((NC * N_PAD, HID), jnp.float32),
    mesh=_mesh,
    scratch_types=[
        pltpu.VMEM((K, CH), jnp.int32),
        pltpu.VMEM((CH, HID), jnp.float32),
        pltpu.VMEM((ZR, HID), jnp.float32),
        pltpu.VMEM_SHARED((N_PAD, HID), jnp.float32),
        pltpu.SemaphoreType.DMA,
    ],
    compiler_params=pltpu.CompilerParams(use_tc_tiling_on_sc=False),
)
def _sc_degree(dstm_hbm, dstt_hbm, out_hbm, dstv, ones_v, zv, acc, sem):
    c = lax.axis_index("c")
    s = lax.axis_index("s")
    wid = c * NS + s

    _zero_acc(zv, acc, s)

    def _fill1(i, carry):
        for u in range(8):
            ones_v[i * 8 + u] = jnp.ones((HID,), jnp.float32)
        return carry

    lax.fori_loop(0, CH // 8, _fill1, None)

    cpm = pltpu.async_copy(dstm_hbm.at[pl.ds(wid * KM, KM)],
                           dstv.at[pl.ds(0, KM)], sem)
    cpt = pltpu.async_copy(dstt_hbm.at[pl.ds(wid * KT, KT)],
                           dstv.at[pl.ds(KM, KT)], sem)
    cpm.wait()
    cpt.wait()

    plsc.subcore_barrier()

    def _chunk(j, carry):
        pltpu.async_copy(ones_v, acc.at[dstv.at[j]], sem, add=True)
        return carry

    lax.fori_loop(0, K, _chunk, None)

    def _drain(j, carry):
        pltpu.make_async_copy(out_hbm.at[pl.ds(0, CH)], ones_v, sem).wait()
        return carry

    lax.fori_loop(0, K, _drain, None)

    plsc.subcore_barrier()
    pltpu.sync_copy(acc.at[pl.ds(s * OW, OW)],
                    out_hbm.at[pl.ds(c * N_PAD + s * OW, OW)])


@functools.partial(
    pl.kernel,
    out_type=jax.ShapeDtypeStruct((NC * N_PAD, HID), jnp.float32),
    mesh=_mesh,
    scratch_types=[
        pltpu.VMEM((K, CH), jnp.int32),
        pltpu.VMEM((K, CH), jnp.int32),
        pltpu.VMEM((NB, CH, HID), jnp.float32),
        pltpu.VMEM((NB, CH, HID), jnp.float32),
        pltpu.VMEM((NB, CH, HID), jnp.float32),
        pltpu.VMEM((NB, CH, HID), jnp.float32),
        pltpu.VMEM((ZR, HID), jnp.float32),
        pltpu.VMEM_SHARED((N_PAD, HID), jnp.float32),
        pltpu.SemaphoreType.DMA,
        pltpu.SemaphoreType.DMA,
        pltpu.SemaphoreType.DMA,
        pltpu.SemaphoreType.DMA,
        pltpu.SemaphoreType.DMA,
        pltpu.SemaphoreType.DMA,
        pltpu.SemaphoreType.DMA,
        pltpu.SemaphoreType.DMA,
    ],
    compiler_params=pltpu.CompilerParams(use_tc_tiling_on_sc=False),
)
def _sc_edge_agg(g_hbm, srcm_hbm, srct_hbm, dstm_hbm, dstt_hbm, out_hbm,
                 srcv, dstv, r0, r1, r2, r3, zv, acc,
                 sg0, sg1, sg2, sg3, ss0, ss1, ss2, ss3):
    c = lax.axis_index("c")
    s = lax.axis_index("s")
    wid = c * NS + s
    rows = (r0, r1, r2, r3)
    semg = (sg0, sg1, sg2, sg3)
    sems = (ss0, ss1, ss2, ss3)

    with jax.named_scope("agg_zero"):
        _zero_acc(zv, acc, s)

    with jax.named_scope("agg_loadidx"):
        cps = [
            pltpu.async_copy(srcm_hbm.at[pl.ds(wid * KM, KM)],
                             srcv.at[pl.ds(0, KM)], sg0),
            pltpu.async_copy(srct_hbm.at[pl.ds(wid * KT, KT)],
                             srcv.at[pl.ds(KM, KT)], sg1),
            pltpu.async_copy(dstm_hbm.at[pl.ds(wid * KM, KM)],
                             dstv.at[pl.ds(0, KM)], sg2),
            pltpu.async_copy(dstt_hbm.at[pl.ds(wid * KT, KT)],
                             dstv.at[pl.ds(KM, KT)], sg3),
        ]
        for cp in cps:
            cp.wait()

    with jax.named_scope("agg_barrier1"):
        plsc.subcore_barrier()

    def _fire_g(g, p):
        for b in range(NB):
            pltpu.async_copy(g_hbm.at[srcv.at[g * NB + b]], rows[p].at[b], semg[p])

    def _wait_g(p):
        for b in range(NB):
            pltpu.make_async_copy(g_hbm.at[pl.ds(0, CH)], rows[p].at[b],
                                  semg[p]).wait()

    def _fire_s(g, p):
        for b in range(NB):
            pltpu.async_copy(rows[p].at[b], acc.at[dstv.at[g * NB + b]],
                             sems[p], add=True)

    def _wait_s(p):
        for b in range(NB):
            pltpu.make_async_copy(g_hbm.at[pl.ds(0, CH)], rows[p].at[b],
                                  sems[p]).wait()

    with jax.named_scope("agg_prologue"):
        _fire_g(0, 0)
        _fire_g(1, 1)

    def _body(t, carry):
        for p in range(4):
            g = 4 * t + p
            q = (p + 2) % 4
            _wait_g(p)
            _fire_s(g, p)
            if p >= 2:
                _wait_s(q)

                @pl.when(t < NG4 - 1)
                def _():
                    _fire_g(g + 2, q)
            else:
                @pl.when(t > 0)
                def _():
                    _wait_s(q)

                _fire_g(g + 2, q)
        return carry

    with jax.named_scope("agg_mainloop"):
        lax.fori_loop(0, NG4, _body, None)
        _wait_s(2)
        _wait_s(3)

    with jax.named_scope("agg_barrier2"):
        plsc.subcore_barrier()
    with jax.named_scope("agg_copyout"):
        pltpu.sync_copy(acc.at[pl.ds(s * OW, OW)],
                        out_hbm.at[pl.ds(c * N_PAD + s * OW, OW)])


def _tc_mm0_body(x_ref, w_ref, h_ref):
    h = jnp.dot(x_ref[...], w_ref[...], preferred_element_type=jnp.float32)
    h_ref[...] = jnp.concatenate(
        [h, jnp.zeros((PADR, 8 * HID), jnp.float32)], axis=0)


def _tc_mm0(xr, W0bd):
    return pl.pallas_call(
        _tc_mm0_body,
        out_shape=jax.ShapeDtypeStruct((NP, 8 * HID), jnp.float32),
    )(xr, W0bd)


def _tc_scale_body(h_ref, degp_ref, g_ref, dinv_ref):
    deg = degp_ref[0] + degp_ref[1] + 1.0
    dinv = lax.rsqrt(jnp.maximum(deg, 1.0))
    g_ref[...] = h_ref[...] * dinv
    dinv_ref[...] = dinv


def _tc_scale(h, degp):
    return pl.pallas_call(
        _tc_scale_body,
        out_shape=[
            jax.ShapeDtypeStruct((NP, 8 * HID), jnp.float32),
            jax.ShapeDtypeStruct((NP, 8 * HID), jnp.float32),
        ],
    )(h, degp)


def _tc_mid_body(p_ref, g_ref, dinv_ref, w_ref, b_ref, gn_ref):
    dinv = dinv_ref[...]
    z = dinv * (p_ref[0] + p_ref[1] + g_ref[...]) + b_ref[...]
    t = jnp.tanh(z)
    gn_ref[...] = jnp.dot(t, w_ref[...], preferred_element_type=jnp.float32) * dinv


def _tc_mid(p, g, dinv, Wbd, bt):
    return pl.pallas_call(
        _tc_mid_body,
        out_shape=jax.ShapeDtypeStruct((NP, 8 * HID), jnp.float32),
    )(p, g, dinv, Wbd, bt)


def _tc_fin_body(p_ref, g_ref, dinv_ref, b_ref, wc_ref, bc_ref, out_ref, emb_ref):
    dinv = dinv_ref[...]
    z = dinv * (p_ref[0] + p_ref[1] + g_ref[...]) + b_ref[...]
    emb = jnp.tanh(z)
    emb_ref[...] = emb
    out_ref[...] = jnp.dot(emb, wc_ref[...], preferred_element_type=jnp.float32) + bc_ref[...]


def _tc_fin(p, g, dinv, b2t, Wcbd, bct):
    return pl.pallas_call(
        _tc_fin_body,
        out_shape=[
            jax.ShapeDtypeStruct((NP, 8 * NCLS), jnp.float32),
            jax.ShapeDtypeStruct((NP, 8 * HID), jnp.float32),
        ],
    )(p, g, dinv, b2t, Wcbd, bct)


def kernel(x, edge_index, W0, b0, W1, b1, W2, b2, Wc, bc):
    src = edge_index[0]
    dst = edge_index[1]
    eye8 = jnp.eye(8, dtype=jnp.float32)

    srcm = src[:EM].reshape(NW * KM, CH)
    dstm = dst[:EM].reshape(NW * KM, CH)
    pad_i = jnp.arange((TT - TREAL) * CH, dtype=jnp.int32)
    srct = jnp.concatenate([src[EM:], pad_i % N]).reshape(TT, CH)
    dstt = jnp.concatenate([dst[EM:], N + pad_i % (N_PAD - N)]).reshape(TT, CH)

    W1bd = jnp.kron(eye8, W1)
    W2bd = jnp.kron(eye8, W2)
    Wcbd = jnp.kron(eye8, Wc)
    b0t = jnp.tile(b0, 8).reshape(1, 8 * HID)
    b1t = jnp.tile(b1, 8).reshape(1, 8 * HID)
    b2t = jnp.tile(b2, 8).reshape(1, 8 * HID)
    bct = jnp.tile(bc, 8).reshape(1, 8 * NCLS)

    xr = x.reshape(N // 8, 8 * IN_DIM)
    W0bd = jnp.kron(eye8, W0)
    h0 = _tc_mm0(xr, W0bd)
    degp = _sc_degree(dstm, dstt).reshape(2, NP, 8 * HID)
    g0, dinv = _tc_scale(h0, degp)

    p0 = _sc_edge_agg(g0.reshape(N_PAD, HID), srcm, srct, dstm, dstt)
    g1 = _tc_mid(p0.reshape(2, NP, 8 * HID), g0, dinv, W1bd, b0t)

    p1 = _sc_edge_agg(g1.reshape(N_PAD, HID), srcm, srct, dstm, dstt)
    g2 = _tc_mid(p1.reshape(2, NP, 8 * HID), g1, dinv, W2bd, b1t)

    p2 = _sc_edge_agg(g2.reshape(N_PAD, HID), srcm, srct, dstm, dstt)
    out_p, emb_p = _tc_fin(p2.reshape(2, NP, 8 * HID), g2, dinv, b2t, Wcbd, bct)
    return (out_p.reshape(N_PAD, NCLS)[:N],
            emb_p.reshape(N_PAD, HID)[:N])


# speedup vs baseline: 2.5748x; 1.0163x over previous
import functools

import jax
import jax.numpy as jnp
from jax import lax
from jax.experimental import pallas as pl
from jax.experimental.pallas import tpu as pltpu
from jax.experimental.pallas import tpu_sc as plsc

N = 10000
E = 320000
IN_DIM = 128
HID = 16
NCLS = 8

NC = 2
NS = 16
NW = NC * NS
CH = 128
KM = 72
KT = 8
K = KM + KT
NB = 5
NG4 = K // (NB * 4)
EM = NW * KM * CH
TT = NW * KT
TREAL = (E - EM) // CH
N_PAD = 10112
ZR = N_PAD // NS
OW = N_PAD // NS

NP = N_PAD // 8
PADR = NP - N // 8

_mesh = plsc.VectorSubcoreMesh(core_axis_name="c", subcore_axis_name="s")


def _zero_acc(zv, acc, s):
    def _fill(i, carry):
        for u in range(8):
            zv[i * 8 + u] = jnp.zeros((HID,), jnp.float32)
        return carry

    lax.fori_loop(0, ZR // 8, _fill, None)
    pltpu.sync_copy(zv, acc.at[pl.ds(s * ZR, ZR)])


@functools.partial(
    pl.kernel,
    out_type=jax.ShapeDtypeStruct((NC * N_PAD, HID), jnp.float32),
    mesh=_mesh,
    scratch_types=[
        pltpu.VMEM((K, CH), jnp.int32),
        pltpu.VMEM((CH, HID), jnp.float32),
        pltpu.VMEM((ZR, HID), jnp.float32),
        pltpu.VMEM_SHARED((N_PAD, HID), jnp.float32),
        pltpu.SemaphoreType.DMA,
    ],
    compiler_params=pltpu.CompilerParams(use_tc_tiling_on_sc=False),
)
def _sc_degree(dstm_hbm, dstt_hbm, out_hbm, dstv, ones_v, zv, acc, sem):
    c = lax.axis_index("c")
    s = lax.axis_index("s")
    wid = c * NS + s

    _zero_acc(zv, acc, s)

    def _fill1(i, carry):
        for u in range(8):
            ones_v[i * 8 + u] = jnp.ones((HID,), jnp.float32)
        return carry

    lax.fori_loop(0, CH // 8, _fill1, None)

    cpm = pltpu.async_copy(dstm_hbm.at[pl.ds(wid * KM, KM)],
                           dstv.at[pl.ds(0, KM)], sem)
    cpt = pltpu.async_copy(dstt_hbm.at[pl.ds(wid * KT, KT)],
                           dstv.at[pl.ds(KM, KT)], sem)
    cpm.wait()
    cpt.wait()

    plsc.subcore_barrier()

    def _chunk(j, carry):
        pltpu.async_copy(ones_v, acc.at[dstv.at[j]], sem, add=True)
        return carry

    lax.fori_loop(0, K, _chunk, None)

    def _drain(j, carry):
        pltpu.make_async_copy(out_hbm.at[pl.ds(0, CH)], ones_v, sem).wait()
        return carry

    lax.fori_loop(0, K, _drain, None)

    plsc.subcore_barrier()
    pltpu.sync_copy(acc.at[pl.ds(s * OW, OW)],
                    out_hbm.at[pl.ds(c * N_PAD + s * OW, OW)])


@functools.partial(
    pl.kernel,
    out_type=jax.ShapeDtypeStruct((NC * N_PAD, HID), jnp.float32),
    mesh=_mesh,
    scratch_types=[
        pltpu.VMEM((K, CH), jnp.int32),
        pltpu.VMEM((K, CH), jnp.int32),
        pltpu.VMEM((NB, CH, HID), jnp.float32),
        pltpu.VMEM((NB, CH, HID), jnp.float32),
        pltpu.VMEM((NB, CH, HID), jnp.float32),
        pltpu.VMEM((NB, CH, HID), jnp.float32),
        pltpu.VMEM((ZR, HID), jnp.float32),
        pltpu.VMEM_SHARED((N_PAD, HID), jnp.float32),
        pltpu.SemaphoreType.DMA,
        pltpu.SemaphoreType.DMA,
        pltpu.SemaphoreType.DMA,
        pltpu.SemaphoreType.DMA,
        pltpu.SemaphoreType.DMA,
        pltpu.SemaphoreType.DMA,
        pltpu.SemaphoreType.DMA,
        pltpu.SemaphoreType.DMA,
    ],
    compiler_params=pltpu.CompilerParams(use_tc_tiling_on_sc=False),
)
def _sc_edge_agg(g_hbm, srcm_hbm, srct_hbm, dstm_hbm, dstt_hbm, out_hbm,
                 srcv, dstv, r0, r1, r2, r3, zv, acc,
                 sg0, sg1, sg2, sg3, ss0, ss1, ss2, ss3):
    c = lax.axis_index("c")
    s = lax.axis_index("s")
    wid = c * NS + s
    rows = (r0, r1, r2, r3)
    semg = (sg0, sg1, sg2, sg3)
    sems = (ss0, ss1, ss2, ss3)

    with jax.named_scope("agg_zero"):
        _zero_acc(zv, acc, s)

    with jax.named_scope("agg_loadidx"):
        cps = [
            pltpu.async_copy(srcm_hbm.at[pl.ds(wid * KM, KM)],
                             srcv.at[pl.ds(0, KM)], sg0),
            pltpu.async_copy(srct_hbm.at[pl.ds(wid * KT, KT)],
                             srcv.at[pl.ds(KM, KT)], sg1),
            pltpu.async_copy(dstm_hbm.at[pl.ds(wid * KM, KM)],
                             dstv.at[pl.ds(0, KM)], sg2),
            pltpu.async_copy(dstt_hbm.at[pl.ds(wid * KT, KT)],
                             dstv.at[pl.ds(KM, KT)], sg3),
        ]
        for cp in cps:
            cp.wait()

    with jax.named_scope("agg_barrier1"):
        plsc.subcore_barrier()

    def _fire_g(g, p):
        for b in range(NB):
            pltpu.async_copy(g_hbm.at[srcv.at[g * NB + b]], rows[p].at[b], semg[p])

    def _wait_g(p):
        for b in range(NB):
            pltpu.make_async_copy(g_hbm.at[pl.ds(0, CH)], rows[p].at[b],
                                  semg[p]).wait()

    def _fire_s(g, p):
        for b in range(NB):
            pltpu.async_copy(rows[p].at[b], acc.at[dstv.at[g * NB + b]],
                             sems[p], add=True)

    def _wait_s(p):
        for b in range(NB):
            pltpu.make_async_copy(g_hbm.at[pl.ds(0, CH)], rows[p].at[b],
                                  sems[p]).wait()

    with jax.named_scope("agg_prologue"):
        _fire_g(0, 0)
        _fire_g(1, 1)

    def _body(t, carry):
        for p in range(4):
            g = 4 * t + p
            q = (p + 2) % 4
            _wait_g(p)
            _fire_s(g, p)
            if p >= 2:
                _wait_s(q)

                @pl.when(t < NG4 - 1)
                def _():
                    _fire_g(g + 2, q)
            else:
                @pl.when(t > 0)
                def _():
                    _wait_s(q)

                _fire_g(g + 2, q)
        return carry

    with jax.named_scope("agg_mainloop"):
        lax.fori_loop(0, NG4, _body, None)
        _wait_s(2)
        _wait_s(3)

    with jax.named_scope("agg_barrier2"):
        plsc.subcore_barrier()
    with jax.named_scope("agg_copyout"):
        pltpu.sync_copy(acc.at[pl.ds(s * OW, OW)],
                        out_hbm.at[pl.ds(c * N_PAD + s * OW, OW)])


def _tc_mm0_body(x_ref, w_ref, h_ref):
    h = jnp.dot(x_ref[...], w_ref[...], preferred_element_type=jnp.float32)
    h_ref[...] = jnp.concatenate(
        [h, jnp.zeros((PADR, 8 * HID), jnp.float32)], axis=0)


def _tc_mm0(xr, W0bd):
    return pl.pallas_call(
        _tc_mm0_body,
        out_shape=jax.ShapeDtypeStruct((NP, 8 * HID), jnp.float32),
    )(xr, W0bd)


def _tc_scale_body(h_ref, degp_ref, g_ref, dinv_ref):
    deg = degp_ref[0] + degp_ref[1] + 1.0
    dinv = lax.rsqrt(jnp.maximum(deg, 1.0))
    g_ref[...] = h_ref[...] * dinv
    dinv_ref[...] = dinv


def _tc_scale(h, degp):
    return pl.pallas_call(
        _tc_scale_body,
        out_shape=[
            jax.ShapeDtypeStruct((NP, 8 * HID), jnp.float32),
            jax.ShapeDtypeStruct((NP, 8 * HID), jnp.float32),
        ],
    )(h, degp)


def _tc_mid_body(p_ref, g_ref, dinv_ref, w_ref, b_ref, gn_ref):
    dinv = dinv_ref[...]
    z = dinv * (p_ref[0] + p_ref[1] + g_ref[...]) + b_ref[...]
    t = jnp.tanh(z)
    gn_ref[...] = jnp.dot(t, w_ref[...], preferred_element_type=jnp.float32) * dinv


def _tc_mid(p, g, dinv, Wbd, bt):
    return pl.pallas_call(
        _tc_mid_body,
        out_shape=jax.ShapeDtypeStruct((NP, 8 * HID), jnp.float32),
    )(p, g, dinv, Wbd, bt)


def _tc_fin_body(p_ref, g_ref, dinv_ref, b_ref, wc_ref, bc_ref, out_ref, emb_ref):
    dinv = dinv_ref[...]
    z = dinv * (p_ref[0] + p_ref[1] + g_ref[...]) + b_ref[...]
    emb = jnp.tanh(z)
    emb_ref[...] = emb
    out_ref[...] = jnp.dot(emb, wc_ref[...], preferred_element_type=jnp.float32) + bc_ref[...]


def _tc_fin(p, g, dinv, b2t, Wcbd, bct):
    return pl.pallas_call(
        _tc_fin_body,
        out_shape=[
            jax.ShapeDtypeStruct((NP, 8 * NCLS), jnp.float32),
            jax.ShapeDtypeStruct((NP, 8 * HID), jnp.float32),
        ],
    )(p, g, dinv, b2t, Wcbd, bct)


def kernel(x, edge_index, W0, b0, W1, b1, W2, b2, Wc, bc):
    src = edge_index[0]
    dst = edge_index[1]
    eye8 = jnp.eye(8, dtype=jnp.float32)

    srcm = src[:EM].reshape(NW * KM, CH)
    dstm = dst[:EM].reshape(NW * KM, CH)
    pad_i = jnp.arange((TT - TREAL) * CH, dtype=jnp.int32)
    srct = jnp.concatenate([src[EM:], pad_i % N]).reshape(TT, CH)
    dstt = jnp.concatenate([dst[EM:], N + pad_i % (N_PAD - N)]).reshape(TT, CH)

    W1bd = jnp.kron(eye8, W1)
    W2bd = jnp.kron(eye8, W2)
    Wcbd = jnp.kron(eye8, Wc)
    b0t = jnp.tile(b0, 8).reshape(1, 8 * HID)
    b1t = jnp.tile(b1, 8).reshape(1, 8 * HID)
    b2t = jnp.tile(b2, 8).reshape(1, 8 * HID)
    bct = jnp.tile(bc, 8).reshape(1, 8 * NCLS)

    xr = x.reshape(N // 8, 8 * IN_DIM)
    W0bd = jnp.kron(eye8, W0)
    h0 = _tc_mm0(xr, W0bd)
    degp = _sc_degree(dstm, dstt).reshape(2, NP, 8 * HID)
    g0, dinv = _tc_scale(h0, degp)

    p0 = _sc_edge_agg(g0.reshape(N_PAD, HID), srcm, srct, dstm, dstt)
    g1 = _tc_mid(p0.reshape(2, NP, 8 * HID), g0, dinv, W1bd, b0t)

    p1 = _sc_edge_agg(g1.reshape(N_PAD, HID), srcm, srct, dstm, dstt)
    g2 = _tc_mid(p1.reshape(2, NP, 8 * HID), g1, dinv, W2bd, b1t)

    p2 = _sc_edge_agg(g2.reshape(N_PAD, HID), srcm, srct, dstm, dstt)
    out_p, emb_p = _tc_fin(p2.reshape(2, NP, 8 * HID), g2, dinv, b2t, Wcbd, bct)
    return (out_p.reshape(N_PAD, NCLS)[:N],
            emb_p.reshape(N_PAD, HID)[:N])
